# baseline (speedup 1.0000x reference)
import functools

import jax
import jax.numpy as jnp
from jax import lax
from jax.experimental import pallas as pl
from jax.experimental.pallas import tpu as pltpu
from jax.experimental.pallas import tpu_sc as plsc

N = 10000
E = 320000
D = 128

NC = 2
NS = 16
NW = NC * NS
EPT = E // NW
CHUNK = 48
NCH = EPT // CHUNK
TAIL = EPT - NCH * CHUNK
SLAB = 624
REM = N - NS * SLAB


def _segsum_body(g_hbm, srcf_hbm, dstf_hbm, out_hbm, agg_sh, sidx, didx,
                 rows0, rows1, rows2, rows3, gsem0, gsem1, gsem2, gsem3,
                 ssem0, ssem1, ssem2, ssem3, isem):
    rows = (rows0, rows1, rows2, rows3)
    gsems = (gsem0, gsem1, gsem2, gsem3)
    ssems = (ssem0, ssem1, ssem2, ssem3)
    c = lax.axis_index("c")
    s = lax.axis_index("s")
    wid = c * NS + s
    base = wid * EPT

    sd = pltpu.async_copy(srcf_hbm.at[pl.ds(base, EPT)], sidx, isem)
    dd = pltpu.async_copy(dstf_hbm.at[pl.ds(base, EPT)], didx, isem)

    zero16 = jnp.zeros((16,), jnp.float32)

    def zfill(i, carry):
        for k in range(D // 16):
            rows0[i, pl.ds(k * 16, 16)] = zero16
        return carry

    lax.fori_loop(0, CHUNK, zfill, 0)
    for q in range(SLAB // CHUNK):
        pltpu.sync_copy(rows0, agg_sh.at[pl.ds(s * SLAB + q * CHUNK, CHUNK)])

    @pl.when(s == NS - 1)
    def _zero_rem():
        pltpu.sync_copy(rows0.at[pl.ds(0, REM)],
                        agg_sh.at[pl.ds(NS * SLAB, REM)])

    sd.wait()
    dd.wait()
    plsc.subcore_barrier()

    def fire_gather(j, b):
        pltpu.async_copy(g_hbm.at[sidx.at[pl.ds(j * CHUNK, CHUNK)]],
                         rows[b], gsems[b])

    def wait_gather(j, b):
        pltpu.make_async_copy(g_hbm.at[sidx.at[pl.ds(j * CHUNK, CHUNK)]],
                              rows[b], gsems[b]).wait()

    def fire_scatter(j, b):
        pltpu.async_copy(rows[b], agg_sh.at[didx.at[pl.ds(j * CHUNK, CHUNK)]],
                         ssems[b], add=True)

    def wait_scatter(j, b):
        pltpu.make_async_copy(rows[b],
                              agg_sh.at[didx.at[pl.ds(j * CHUNK, CHUNK)]],
                              ssems[b]).wait()

    fire_gather(0, 0)
    fire_gather(1, 1)
    wait_gather(0, 0)
    fire_scatter(0, 0)
    fire_gather(2, 2)
    wait_gather(1, 1)
    fire_scatter(1, 1)
    fire_gather(3, 3)

    def steady(g, carry):
        jb = 4 * g + 2
        for k in range(4):
            j = jb + k
            b = (2 + k) % 4
            wait_gather(j, b)
            fire_scatter(j, b)
            wait_scatter(j - 2, k % 4)
            fire_gather(j + 2, k % 4)
        return carry

    lax.fori_loop(0, 51, steady, 0)
    for j in range(206, 208):
        b = j % 4
        wait_gather(j, b)
        fire_scatter(j, b)
        wait_scatter(j - 2, (j - 2) % 4)

    tb = rows0.at[pl.ds(0, TAIL)]
    tsrc = sidx.at[pl.ds(NCH * CHUNK, TAIL)]
    tdst = didx.at[pl.ds(NCH * CHUNK, TAIL)]
    pltpu.async_copy(g_hbm.at[tsrc], tb, gsems[0])
    pltpu.make_async_copy(g_hbm.at[tsrc], tb, gsems[0]).wait()
    pltpu.async_copy(tb, agg_sh.at[tdst], ssems[0], add=True)
    wait_scatter(206, 2)
    wait_scatter(207, 3)
    pltpu.make_async_copy(tb, agg_sh.at[tdst], ssems[0]).wait()
    plsc.subcore_barrier()

    pltpu.sync_copy(agg_sh.at[pl.ds(s * SLAB, SLAB)],
                    out_hbm.at[c, pl.ds(s * SLAB, SLAB)])

    @pl.when(s == NS - 1)
    def _drain_rem():
        pltpu.sync_copy(agg_sh.at[pl.ds(NS * SLAB, REM)],
                        out_hbm.at[c, pl.ds(NS * SLAB, REM)])


def _make_segsum():
    mesh = plsc.VectorSubcoreMesh(core_axis_name="c", subcore_axis_name="s")
    scratch = [
        pltpu.VMEM_SHARED((N, D), jnp.float32),
        pltpu.VMEM((EPT,), jnp.int32),
        pltpu.VMEM((EPT,), jnp.int32),
        pltpu.VMEM((CHUNK, D), jnp.float32),
        pltpu.VMEM((CHUNK, D), jnp.float32),
        pltpu.VMEM((CHUNK, D), jnp.float32),
        pltpu.VMEM((CHUNK, D), jnp.float32),
    ] + [pltpu.SemaphoreType.DMA] * 9
    return pl.kernel(
        _segsum_body,
        out_type=jax.ShapeDtypeStruct((NC, N, D), jnp.float32),
        mesh=mesh,
        scratch_types=scratch,
    )


def _mm_body(x_ref, w_ref, o_ref):
    o_ref[...] = lax.dot_general(
        x_ref[...], w_ref[...], (((1,), (1,)), ((), ())),
        preferred_element_type=jnp.float32)


def _mm(x, w):
    return pl.pallas_call(
        _mm_body,
        grid=(10,),
        in_specs=[
            pl.BlockSpec((N // 10, D), lambda i: (i, 0)),
            pl.BlockSpec((D, D), lambda i: (0, 0)),
        ],
        out_specs=pl.BlockSpec((N // 10, D), lambda i: (i, 0)),
        out_shape=jax.ShapeDtypeStruct((N, D), jnp.float32),
    )(x, w)


def _combine_mm_body(scale_ref, g_ref, agg_ref, b_ref, w_ref, o_ref):
    z = (scale_ref[0] * g_ref[...] + agg_ref[0] + agg_ref[1]
         + b_ref[...][None, :])
    o_ref[...] = lax.dot_general(
        z, w_ref[...], (((1,), (1,)), ((), ())),
        preferred_element_type=jnp.float32)


def _combine_mm(scale, g, agg, b, w):
    return pl.pallas_call(
        _combine_mm_body,
        grid=(10,),
        in_specs=[
            pl.BlockSpec(memory_space=pltpu.SMEM),
            pl.BlockSpec((N // 10, D), lambda i: (i, 0)),
            pl.BlockSpec((NC, N // 10, D), lambda i: (0, i, 0)),
            pl.BlockSpec((D,), lambda i: (0,)),
            pl.BlockSpec((D, D), lambda i: (0, 0)),
        ],
        out_specs=pl.BlockSpec((N // 10, D), lambda i: (i, 0)),
        out_shape=jax.ShapeDtypeStruct((N, D), jnp.float32),
    )(scale, g, agg, b, w)


def _combine_body(scale_ref, g_ref, agg_ref, b_ref, o_ref):
    o_ref[...] = (scale_ref[0] * g_ref[...] + agg_ref[0] + agg_ref[1]
                  + b_ref[...][None, :])


def _combine(scale, g, agg, b):
    return pl.pallas_call(
        _combine_body,
        grid=(10,),
        in_specs=[
            pl.BlockSpec(memory_space=pltpu.SMEM),
            pl.BlockSpec((N // 10, D), lambda i: (i, 0)),
            pl.BlockSpec((NC, N // 10, D), lambda i: (0, i, 0)),
            pl.BlockSpec((D,), lambda i: (0,)),
        ],
        out_specs=pl.BlockSpec((N // 10, D), lambda i: (i, 0)),
        out_shape=jax.ShapeDtypeStruct((N, D), jnp.float32),
    )(scale, g, agg, b)


_segsum = _make_segsum()


def kernel(feats, edge_index, W1, b1, W2, b2, eps1, eps2):
    srcf = edge_index[0]
    dstf = edge_index[1]
    scale1 = (1.0 + eps1).reshape(1)
    scale2 = (1.0 + eps2).reshape(1)
    g1 = _mm(feats, W1)
    agg1 = _segsum(g1, srcf, dstf)
    g2 = _combine_mm(scale1, g1, agg1, b1, W2)
    agg2 = _segsum(g2, srcf, dstf)
    return _combine(scale2, g2, agg2, b2)


# speedup vs baseline: 1.2357x; 1.2357x over previous
import functools

import jax
import jax.numpy as jnp
from jax import lax
from jax.experimental import pallas as pl
from jax.experimental.pallas import tpu as pltpu
from jax.experimental.pallas import tpu_sc as plsc

N = 10000
E = 320000
D = 128

NC = 2
NS = 16
NW = NC * NS
EPT = E // NW
CHUNK = 80
NCH = EPT // CHUNK
SLAB = 624
REM = N - NS * SLAB


def _segsum_body(g_hbm, srcf_hbm, dstf_hbm, out_hbm, agg_sh, sidx, didx,
                 rows0, rows1, rows2, gsem0, gsem1, gsem2,
                 ssem0, ssem1, ssem2, isem):
    rows = (rows0, rows1, rows2)
    gsems = (gsem0, gsem1, gsem2)
    ssems = (ssem0, ssem1, ssem2)
    c = lax.axis_index("c")
    s = lax.axis_index("s")
    wid = c * NS + s
    base = wid * EPT

    sd = pltpu.async_copy(srcf_hbm.at[pl.ds(base, EPT)], sidx, isem)
    dd = pltpu.async_copy(dstf_hbm.at[pl.ds(base, EPT)], didx, isem)

    zero16 = jnp.zeros((16,), jnp.float32)

    def zfill(i, carry):
        for k in range(D // 16):
            rows0[i, pl.ds(k * 16, 16)] = zero16
        return carry

    lax.fori_loop(0, CHUNK, zfill, 0)
    for q in range(SLAB // CHUNK):
        pltpu.sync_copy(rows0, agg_sh.at[pl.ds(s * SLAB + q * CHUNK, CHUNK)])
    zrem = SLAB - (SLAB // CHUNK) * CHUNK
    pltpu.sync_copy(rows0.at[pl.ds(0, zrem)],
                    agg_sh.at[pl.ds(s * SLAB + SLAB - zrem, zrem)])

    @pl.when(s == NS - 1)
    def _zero_rem():
        pltpu.sync_copy(rows0.at[pl.ds(0, REM)],
                        agg_sh.at[pl.ds(NS * SLAB, REM)])

    sd.wait()
    dd.wait()
    plsc.subcore_barrier()

    def fire_gather(j, b):
        pltpu.async_copy(g_hbm.at[sidx.at[pl.ds(j * CHUNK, CHUNK)]],
                         rows[b], gsems[b])

    def wait_gather(j, b):
        pltpu.make_async_copy(g_hbm.at[sidx.at[pl.ds(j * CHUNK, CHUNK)]],
                              rows[b], gsems[b]).wait()

    def fire_scatter(j, b):
        pltpu.async_copy(rows[b], agg_sh.at[didx.at[pl.ds(j * CHUNK, CHUNK)]],
                         ssems[b], add=True)

    def wait_scatter(j, b):
        pltpu.make_async_copy(rows[b],
                              agg_sh.at[didx.at[pl.ds(j * CHUNK, CHUNK)]],
                              ssems[b]).wait()

    fire_gather(0, 0)
    fire_gather(1, 1)
    wait_gather(0, 0)
    fire_scatter(0, 0)
    fire_gather(2, 2)

    def steady(g, carry):
        jb = 3 * g + 1
        for k in range(3):
            j = jb + k
            b = (1 + k) % 3
            wait_gather(j, b)
            fire_scatter(j, b)
            wait_scatter(j - 1, k % 3)
            fire_gather(j + 2, k % 3)
        return carry

    lax.fori_loop(0, 40, steady, 0)
    for j in range(121, 123):
        b = j % 3
        wait_gather(j, b)
        fire_scatter(j, b)
        wait_scatter(j - 1, (j - 1) % 3)
        fire_gather(j + 2, (j - 1) % 3)
    for j in range(123, 125):
        b = j % 3
        wait_gather(j, b)
        fire_scatter(j, b)
        wait_scatter(j - 1, (j - 1) % 3)
    wait_scatter(NCH - 1, (NCH - 1) % 3)
    plsc.subcore_barrier()

    pltpu.sync_copy(agg_sh.at[pl.ds(s * SLAB, SLAB)],
                    out_hbm.at[c, pl.ds(s * SLAB, SLAB)])

    @pl.when(s == NS - 1)
    def _drain_rem():
        pltpu.sync_copy(agg_sh.at[pl.ds(NS * SLAB, REM)],
                        out_hbm.at[c, pl.ds(NS * SLAB, REM)])


def _make_segsum():
    mesh = plsc.VectorSubcoreMesh(core_axis_name="c", subcore_axis_name="s")
    scratch = [
        pltpu.VMEM_SHARED((N, D), jnp.float32),
        pltpu.VMEM((EPT,), jnp.int32),
        pltpu.VMEM((EPT,), jnp.int32),
        pltpu.VMEM((CHUNK, D), jnp.float32),
        pltpu.VMEM((CHUNK, D), jnp.float32),
        pltpu.VMEM((CHUNK, D), jnp.float32),
    ] + [pltpu.SemaphoreType.DMA] * 7
    return pl.kernel(
        _segsum_body,
        out_type=jax.ShapeDtypeStruct((NC, N, D), jnp.float32),
        mesh=mesh,
        scratch_types=scratch,
    )


def _mm_body(x_ref, w_ref, o_ref):
    o_ref[...] = lax.dot_general(
        x_ref[...], w_ref[...], (((1,), (1,)), ((), ())),
        preferred_element_type=jnp.float32)


def _mm(x, w):
    return pl.pallas_call(
        _mm_body,
        grid=(10,),
        in_specs=[
            pl.BlockSpec((N // 10, D), lambda i: (i, 0)),
            pl.BlockSpec((D, D), lambda i: (0, 0)),
        ],
        out_specs=pl.BlockSpec((N // 10, D), lambda i: (i, 0)),
        out_shape=jax.ShapeDtypeStruct((N, D), jnp.float32),
    )(x, w)


def _combine_mm_body(scale_ref, g_ref, agg_ref, b_ref, w_ref, o_ref):
    z = (scale_ref[0] * g_ref[...] + agg_ref[0] + agg_ref[1]
         + b_ref[...][None, :])
    o_ref[...] = lax.dot_general(
        z, w_ref[...], (((1,), (1,)), ((), ())),
        preferred_element_type=jnp.float32)


def _combine_mm(scale, g, agg, b, w):
    return pl.pallas_call(
        _combine_mm_body,
        grid=(10,),
        in_specs=[
            pl.BlockSpec(memory_space=pltpu.SMEM),
            pl.BlockSpec((N // 10, D), lambda i: (i, 0)),
            pl.BlockSpec((NC, N // 10, D), lambda i: (0, i, 0)),
            pl.BlockSpec((D,), lambda i: (0,)),
            pl.BlockSpec((D, D), lambda i: (0, 0)),
        ],
        out_specs=pl.BlockSpec((N // 10, D), lambda i: (i, 0)),
        out_shape=jax.ShapeDtypeStruct((N, D), jnp.float32),
    )(scale, g, agg, b, w)


def _combine_body(scale_ref, g_ref, agg_ref, b_ref, o_ref):
    o_ref[...] = (scale_ref[0] * g_ref[...] + agg_ref[0] + agg_ref[1]
                  + b_ref[...][None, :])


def _combine(scale, g, agg, b):
    return pl.pallas_call(
        _combine_body,
        grid=(10,),
        in_specs=[
            pl.BlockSpec(memory_space=pltpu.SMEM),
            pl.BlockSpec((N // 10, D), lambda i: (i, 0)),
            pl.BlockSpec((NC, N // 10, D), lambda i: (0, i, 0)),
            pl.BlockSpec((D,), lambda i: (0,)),
        ],
        out_specs=pl.BlockSpec((N // 10, D), lambda i: (i, 0)),
        out_shape=jax.ShapeDtypeStruct((N, D), jnp.float32),
    )(scale, g, agg, b)


_segsum = _make_segsum()


def kernel(feats, edge_index, W1, b1, W2, b2, eps1, eps2):
    srcf = edge_index[0]
    dstf = edge_index[1]
    scale1 = (1.0 + eps1).reshape(1)
    scale2 = (1.0 + eps2).reshape(1)
    g1 = _mm(feats, W1)
    agg1 = _segsum(g1, srcf, dstf)
    g2 = _combine_mm(scale1, g1, agg1, b1, W2)
    agg2 = _segsum(g2, srcf, dstf)
    return _combine(scale2, g2, agg2, b2)


# speedup vs baseline: 1.2790x; 1.0351x over previous
import functools

import jax
import jax.numpy as jnp
from jax import lax
from jax.experimental import pallas as pl
from jax.experimental.pallas import tpu as pltpu
from jax.experimental.pallas import tpu_sc as plsc

N = 10000
E = 320000
D = 128

NC = 2
NS = 16
NW = NC * NS
EPT = E // NW
CHUNK = 80
NCH = EPT // CHUNK
SLAB = 624
REM = N - NS * SLAB


def _segsum_body(g_hbm, srcf_hbm, dstf_hbm, out_hbm, agg_sh, sidx, didx,
                 rows0, rows1, rows2, gsem0, gsem1, gsem2,
                 ssem0, ssem1, ssem2, isem):
    rows = (rows0, rows1, rows2)
    gsems = (gsem0, gsem1, gsem2)
    ssems = (ssem0, ssem1, ssem2)
    c = lax.axis_index("c")
    s = lax.axis_index("s")
    wid = c * NS + s
    base = wid * EPT

    sd = pltpu.async_copy(srcf_hbm.at[pl.ds(base, EPT)], sidx, isem)
    dd = pltpu.async_copy(dstf_hbm.at[pl.ds(base, EPT)], didx, isem)

    zero16 = jnp.zeros((16,), jnp.float32)

    def zfill(i, carry):
        for k in range(D // 16):
            rows0[i, pl.ds(k * 16, 16)] = zero16
        return carry

    lax.fori_loop(0, CHUNK, zfill, 0)
    for q in range(SLAB // CHUNK):
        pltpu.sync_copy(rows0, agg_sh.at[pl.ds(s * SLAB + q * CHUNK, CHUNK)])
    zrem = SLAB - (SLAB // CHUNK) * CHUNK
    pltpu.sync_copy(rows0.at[pl.ds(0, zrem)],
                    agg_sh.at[pl.ds(s * SLAB + SLAB - zrem, zrem)])

    @pl.when(s == NS - 1)
    def _zero_rem():
        pltpu.sync_copy(rows0.at[pl.ds(0, REM)],
                        agg_sh.at[pl.ds(NS * SLAB, REM)])

    sd.wait()
    dd.wait()
    plsc.subcore_barrier()

    def fire_gather(j, b):
        pltpu.async_copy(g_hbm.at[sidx.at[pl.ds(j * CHUNK, CHUNK)]],
                         rows[b], gsems[b])

    def wait_gather(j, b):
        pltpu.make_async_copy(g_hbm.at[sidx.at[pl.ds(j * CHUNK, CHUNK)]],
                              rows[b], gsems[b]).wait()

    def fire_scatter(j, b):
        pltpu.async_copy(rows[b], agg_sh.at[didx.at[pl.ds(j * CHUNK, CHUNK)]],
                         ssems[b], add=True)

    def wait_scatter(j, b):
        pltpu.make_async_copy(rows[b],
                              agg_sh.at[didx.at[pl.ds(j * CHUNK, CHUNK)]],
                              ssems[b]).wait()

    fire_gather(0, 0)
    fire_gather(1, 1)
    wait_gather(0, 0)
    fire_scatter(0, 0)
    fire_gather(2, 2)

    def steady(g, carry):
        jb = 3 * g + 1
        for k in range(3):
            j = jb + k
            b = (1 + k) % 3
            wait_gather(j, b)
            fire_scatter(j, b)
            wait_scatter(j - 1, k % 3)
            fire_gather(j + 2, k % 3)
        return carry

    lax.fori_loop(0, 40, steady, 0)
    for j in range(121, 123):
        b = j % 3
        wait_gather(j, b)
        fire_scatter(j, b)
        wait_scatter(j - 1, (j - 1) % 3)
        fire_gather(j + 2, (j - 1) % 3)
    for j in range(123, 125):
        b = j % 3
        wait_gather(j, b)
        fire_scatter(j, b)
        wait_scatter(j - 1, (j - 1) % 3)
    wait_scatter(NCH - 1, (NCH - 1) % 3)
    plsc.subcore_barrier()

    pltpu.sync_copy(agg_sh.at[pl.ds(s * SLAB, SLAB)],
                    out_hbm.at[c, pl.ds(s * SLAB, SLAB)])

    @pl.when(s == NS - 1)
    def _drain_rem():
        pltpu.sync_copy(agg_sh.at[pl.ds(NS * SLAB, REM)],
                        out_hbm.at[c, pl.ds(NS * SLAB, REM)])


def _make_segsum():
    mesh = plsc.VectorSubcoreMesh(core_axis_name="c", subcore_axis_name="s")
    scratch = [
        pltpu.VMEM_SHARED((N, D), jnp.float32),
        pltpu.VMEM((EPT,), jnp.int32),
        pltpu.VMEM((EPT,), jnp.int32),
        pltpu.VMEM((CHUNK, D), jnp.float32),
        pltpu.VMEM((CHUNK, D), jnp.float32),
        pltpu.VMEM((CHUNK, D), jnp.float32),
    ] + [pltpu.SemaphoreType.DMA] * 7
    return pl.kernel(
        _segsum_body,
        out_type=jax.ShapeDtypeStruct((NC, N, D), jnp.float32),
        mesh=mesh,
        scratch_types=scratch,
    )


def _cmm_body(scale_ref, h_ref, agg_ref, w_ref, b_ref, o_ref):
    z = scale_ref[0] * h_ref[...] + agg_ref[0] + agg_ref[1]
    o_ref[...] = lax.dot_general(
        z, w_ref[...], (((1,), (1,)), ((), ())),
        preferred_element_type=jnp.float32) + b_ref[...][None, :]


def _cmm(scale, h, agg, w, b):
    return pl.pallas_call(
        _cmm_body,
        grid=(10,),
        in_specs=[
            pl.BlockSpec(memory_space=pltpu.SMEM),
            pl.BlockSpec((N // 10, D), lambda i: (i, 0)),
            pl.BlockSpec((NC, N // 10, D), lambda i: (0, i, 0)),
            pl.BlockSpec((D, D), lambda i: (0, 0)),
            pl.BlockSpec((D,), lambda i: (0,)),
        ],
        out_specs=pl.BlockSpec((N // 10, D), lambda i: (i, 0)),
        out_shape=jax.ShapeDtypeStruct((N, D), jnp.float32),
    )(scale, h, agg, w, b)


_segsum = _make_segsum()


def kernel(feats, edge_index, W1, b1, W2, b2, eps1, eps2):
    srcf = edge_index[0]
    dstf = edge_index[1]
    scale1 = (1.0 + eps1).reshape(1)
    scale2 = (1.0 + eps2).reshape(1)
    aggf = _segsum(feats, srcf, dstf)
    h1 = _cmm(scale1, feats, aggf, W1, b1)
    aggh = _segsum(h1, srcf, dstf)
    return _cmm(scale2, h1, aggh, W2, b2)
